# baseline (speedup 1.0000x reference)
import functools

import jax
import jax.numpy as jnp
from jax import lax
from jax.experimental import pallas as pl
from jax.experimental.pallas import tpu as pltpu
from jax.experimental.pallas import tpu_sc as plsc

_CHUNKS = (36, 126, 126, 102, 12)


def _sc_gather(table, idx_flat, ch):
    n = idx_flat.shape[0]
    d = table.shape[1]
    info = plsc.get_sparse_core_info()
    nc = info.num_cores
    nw = nc * info.num_subcores
    per_w = n // nw
    n_ch = per_w // ch
    tail = per_w - n_ch * ch
    assert per_w * nw == n and n_ch >= 1 and ch % 8 == 0 and tail % 8 == 0

    mesh = plsc.VectorSubcoreMesh(core_axis_name="c", subcore_axis_name="s")

    @functools.partial(
        pl.kernel,
        mesh=mesh,
        out_type=jax.ShapeDtypeStruct((n, d), jnp.float32),
        scratch_types=[
            pltpu.VMEM((per_w,), jnp.int32),
            pltpu.VMEM((2, ch, d), jnp.float32),
            pltpu.SemaphoreType.DMA,
            pltpu.SemaphoreType.DMA,
        ],
    )
    def k(table_hbm, idx_hbm, out_hbm, idx_v, buf, gsem, ssem):
        wid = lax.axis_index("s") * nc + lax.axis_index("c")
        base = wid * per_w
        pltpu.sync_copy(idx_hbm.at[pl.ds(base, per_w)], idx_v)

        def start_gather(c, slot):
            pltpu.async_copy(
                table_hbm.at[idx_v.at[pl.ds(c * ch, ch)]], buf.at[slot], gsem)

        def wait_gather(c, slot):
            pltpu.make_async_copy(
                table_hbm.at[idx_v.at[pl.ds(c * ch, ch)]], buf.at[slot],
                gsem).wait()

        def start_scatter(c, slot):
            pltpu.async_copy(
                buf.at[slot], out_hbm.at[pl.ds(base + c * ch, ch)], ssem)

        def wait_scatter():
            pltpu.make_async_copy(
                buf.at[0], out_hbm.at[pl.ds(base, ch)], ssem).wait()

        def start_gather_tail(slot):
            pltpu.async_copy(
                table_hbm.at[idx_v.at[pl.ds(n_ch * ch, tail)]],
                buf.at[slot].at[pl.ds(0, tail)], gsem)

        def wait_gather_tail(slot):
            pltpu.make_async_copy(
                table_hbm.at[idx_v.at[pl.ds(n_ch * ch, tail)]],
                buf.at[slot].at[pl.ds(0, tail)], gsem).wait()

        def start_scatter_tail(slot):
            pltpu.async_copy(
                buf.at[slot].at[pl.ds(0, tail)],
                out_hbm.at[pl.ds(base + n_ch * ch, tail)], ssem)

        def wait_scatter_tail(slot):
            pltpu.make_async_copy(
                buf.at[slot].at[pl.ds(0, tail)],
                out_hbm.at[pl.ds(base + n_ch * ch, tail)], ssem).wait()

        start_gather(0, 0)

        def step(c):
            slot = lax.rem(c, 2)
            wait_gather(c, slot)
            @pl.when(c >= 1)
            def _():
                wait_scatter()

            @pl.when(c + 1 < n_ch)
            def _():
                start_gather(c + 1, 1 - slot)

            if tail:
                @pl.when(c + 1 == n_ch)
                def _():
                    start_gather_tail(1 - slot)

            start_scatter(c, slot)

        pl.loop(0, n_ch)(step)
        if tail:
            slot_t = n_ch % 2
            wait_gather_tail(slot_t)
            wait_scatter()
            start_scatter_tail(slot_t)
            wait_scatter_tail(slot_t)
        else:
            wait_scatter()

    return k(table, idx_flat)


def _tc_ln_chunk(out_prev, tok_t, pos_chunk, seg_table, gamma, beta,
                 off, max_sent, s_total, sb):
    s_c, b, d = tok_t.shape
    assert s_c % sb == 0 and off % sb == 0

    def body(*refs):
        if out_prev is None:
            tok_ref, pos_ref, seg_ref, g_ref, b_ref, o_ref = refs
        else:
            _, tok_ref, pos_ref, seg_ref, g_ref, b_ref, o_ref = refs
        i = pl.program_id(0)
        h = tok_ref[...] + pos_ref[...]
        srow = lax.broadcasted_iota(jnp.int32, (sb, 1, 1), 0) + i * sb + off
        segv = jnp.where(srow < max_sent + 1, seg_ref[0][None, None, :],
                         seg_ref[1][None, None, :])
        h = h + segv
        mean = jnp.mean(h, axis=-1, keepdims=True)
        c = h - mean
        var = jnp.mean(c * c, axis=-1, keepdims=True)
        o_ref[...] = (c * lax.rsqrt(var + 1e-5)) * g_ref[...] + b_ref[...]

    in_specs = [
        pl.BlockSpec((sb, b, d), lambda i: (i, 0, 0)),
        pl.BlockSpec((sb, 1, d), lambda i: (i, 0, 0)),
        pl.BlockSpec((2, d), lambda i: (0, 0)),
        pl.BlockSpec((d,), lambda i: (0,)),
        pl.BlockSpec((d,), lambda i: (0,)),
    ]
    args = [tok_t, pos_chunk.reshape(s_c, 1, d), seg_table, gamma, beta]
    aliases = {}
    if out_prev is not None:
        in_specs = [pl.BlockSpec(memory_space=pl.ANY)] + in_specs
        args = [out_prev] + args
        aliases = {0: 0}

    return pl.pallas_call(
        body,
        grid=(s_c // sb,),
        in_specs=in_specs,
        out_specs=pl.BlockSpec(
            (sb, b, d), lambda i, _o=off // sb: (i + _o, 0, 0)),
        out_shape=jax.ShapeDtypeStruct((s_total, b, d), jnp.float32),
        input_output_aliases=aliases,
    )(*args)


def kernel(x, token_table, pos_table, seg_table, gamma, beta):
    b, s = x.shape
    d = token_table.shape[1]
    max_sent = (s - 2) // 2
    idx_t = x.T.astype(jnp.int32).reshape(-1)
    assert sum(_CHUNKS) == s

    ch = 192
    sb = 6

    out = None
    off = 0
    for s_c in _CHUNKS:
        tok = _sc_gather(token_table, idx_t[off * b:(off + s_c) * b], ch)
        out = _tc_ln_chunk(out, tok.reshape(s_c, b, d),
                           pos_table[off:off + s_c], seg_table,
                           gamma, beta, off, max_sent, s, sb)
        off += s_c
    return jnp.transpose(out, (1, 0, 2))


# speedup vs baseline: 1.0022x; 1.0022x over previous
import functools

import jax
import jax.numpy as jnp
from jax import lax
from jax.experimental import pallas as pl
from jax.experimental.pallas import tpu as pltpu
from jax.experimental.pallas import tpu_sc as plsc

_CHUNKS = (66, 168, 168)


def _sc_gather(table, idx_flat, ch):
    n = idx_flat.shape[0]
    d = table.shape[1]
    info = plsc.get_sparse_core_info()
    nc = info.num_cores
    nw = nc * info.num_subcores
    per_w = n // nw
    n_ch = per_w // ch
    tail = per_w - n_ch * ch
    assert per_w * nw == n and n_ch >= 1 and ch % 8 == 0 and tail % 8 == 0

    mesh = plsc.VectorSubcoreMesh(core_axis_name="c", subcore_axis_name="s")

    @functools.partial(
        pl.kernel,
        mesh=mesh,
        out_type=jax.ShapeDtypeStruct((n, d), jnp.float32),
        scratch_types=[
            pltpu.VMEM((per_w,), jnp.int32),
            pltpu.VMEM((2, ch, d), jnp.float32),
            pltpu.SemaphoreType.DMA,
            pltpu.SemaphoreType.DMA,
        ],
    )
    def k(table_hbm, idx_hbm, out_hbm, idx_v, buf, gsem, ssem):
        wid = lax.axis_index("s") * nc + lax.axis_index("c")
        base = wid * per_w
        pltpu.sync_copy(idx_hbm.at[pl.ds(base, per_w)], idx_v)

        def start_gather(c, slot):
            pltpu.async_copy(
                table_hbm.at[idx_v.at[pl.ds(c * ch, ch)]], buf.at[slot], gsem)

        def wait_gather(c, slot):
            pltpu.make_async_copy(
                table_hbm.at[idx_v.at[pl.ds(c * ch, ch)]], buf.at[slot],
                gsem).wait()

        def start_scatter(c, slot):
            pltpu.async_copy(
                buf.at[slot], out_hbm.at[pl.ds(base + c * ch, ch)], ssem)

        def wait_scatter():
            pltpu.make_async_copy(
                buf.at[0], out_hbm.at[pl.ds(base, ch)], ssem).wait()

        def start_gather_tail(slot):
            pltpu.async_copy(
                table_hbm.at[idx_v.at[pl.ds(n_ch * ch, tail)]],
                buf.at[slot].at[pl.ds(0, tail)], gsem)

        def wait_gather_tail(slot):
            pltpu.make_async_copy(
                table_hbm.at[idx_v.at[pl.ds(n_ch * ch, tail)]],
                buf.at[slot].at[pl.ds(0, tail)], gsem).wait()

        def start_scatter_tail(slot):
            pltpu.async_copy(
                buf.at[slot].at[pl.ds(0, tail)],
                out_hbm.at[pl.ds(base + n_ch * ch, tail)], ssem)

        def wait_scatter_tail(slot):
            pltpu.make_async_copy(
                buf.at[slot].at[pl.ds(0, tail)],
                out_hbm.at[pl.ds(base + n_ch * ch, tail)], ssem).wait()

        start_gather(0, 0)

        def step(c):
            slot = lax.rem(c, 2)
            wait_gather(c, slot)
            @pl.when(c >= 1)
            def _():
                wait_scatter()

            @pl.when(c + 1 < n_ch)
            def _():
                start_gather(c + 1, 1 - slot)

            if tail:
                @pl.when(c + 1 == n_ch)
                def _():
                    start_gather_tail(1 - slot)

            start_scatter(c, slot)

        pl.loop(0, n_ch)(step)
        if tail:
            slot_t = n_ch % 2
            wait_gather_tail(slot_t)
            wait_scatter()
            start_scatter_tail(slot_t)
            wait_scatter_tail(slot_t)
        else:
            wait_scatter()

    return k(table, idx_flat)


def _tc_ln_chunk(out_prev, tok_t, pos_chunk, seg_table, gamma, beta,
                 off, max_sent, s_total, sb):
    s_c, b, d = tok_t.shape
    assert s_c % sb == 0 and off % sb == 0

    def body(*refs):
        if out_prev is None:
            tok_ref, pos_ref, seg_ref, g_ref, b_ref, o_ref = refs
        else:
            _, tok_ref, pos_ref, seg_ref, g_ref, b_ref, o_ref = refs
        i = pl.program_id(0)
        h = tok_ref[...] + pos_ref[...]
        srow = lax.broadcasted_iota(jnp.int32, (sb, 1, 1), 0) + i * sb + off
        segv = jnp.where(srow < max_sent + 1, seg_ref[0][None, None, :],
                         seg_ref[1][None, None, :])
        h = h + segv
        mean = jnp.mean(h, axis=-1, keepdims=True)
        c = h - mean
        var = jnp.mean(c * c, axis=-1, keepdims=True)
        o_ref[...] = (c * lax.rsqrt(var + 1e-5)) * g_ref[...] + b_ref[...]

    in_specs = [
        pl.BlockSpec((sb, b, d), lambda i: (i, 0, 0)),
        pl.BlockSpec((sb, 1, d), lambda i: (i, 0, 0)),
        pl.BlockSpec((2, d), lambda i: (0, 0)),
        pl.BlockSpec((d,), lambda i: (0,)),
        pl.BlockSpec((d,), lambda i: (0,)),
    ]
    args = [tok_t, pos_chunk.reshape(s_c, 1, d), seg_table, gamma, beta]
    aliases = {}
    if out_prev is not None:
        in_specs = [pl.BlockSpec(memory_space=pl.ANY)] + in_specs
        args = [out_prev] + args
        aliases = {0: 0}

    return pl.pallas_call(
        body,
        grid=(s_c // sb,),
        in_specs=in_specs,
        out_specs=pl.BlockSpec(
            (sb, b, d), lambda i, _o=off // sb: (i + _o, 0, 0)),
        out_shape=jax.ShapeDtypeStruct((s_total, b, d), jnp.float32),
        input_output_aliases=aliases,
    )(*args)


def kernel(x, token_table, pos_table, seg_table, gamma, beta):
    b, s = x.shape
    d = token_table.shape[1]
    max_sent = (s - 2) // 2
    idx_t = x.T.astype(jnp.int32).reshape(-1)
    assert sum(_CHUNKS) == s

    ch = 192
    sb = 6

    out = None
    off = 0
    for s_c in _CHUNKS:
        tok = _sc_gather(token_table, idx_t[off * b:(off + s_c) * b], ch)
        out = _tc_ln_chunk(out, tok.reshape(s_c, b, d),
                           pos_table[off:off + s_c], seg_table,
                           gamma, beta, off, max_sent, s, sb)
        off += s_c
    return jnp.transpose(out, (1, 0, 2))


# speedup vs baseline: 1.0183x; 1.0160x over previous
import functools

import jax
import jax.numpy as jnp
from jax import lax
from jax.experimental import pallas as pl
from jax.experimental.pallas import tpu as pltpu
from jax.experimental.pallas import tpu_sc as plsc

_CHUNKS = (36, 120, 120, 126)


def _sc_gather(table, idx_flat, ch):
    n = idx_flat.shape[0]
    d = table.shape[1]
    info = plsc.get_sparse_core_info()
    nc = info.num_cores
    nw = nc * info.num_subcores
    per_w = n // nw
    n_ch = per_w // ch
    tail = per_w - n_ch * ch
    assert per_w * nw == n and n_ch >= 1 and ch % 8 == 0 and tail % 8 == 0

    mesh = plsc.VectorSubcoreMesh(core_axis_name="c", subcore_axis_name="s")

    @functools.partial(
        pl.kernel,
        mesh=mesh,
        out_type=jax.ShapeDtypeStruct((n, d), jnp.float32),
        scratch_types=[
            pltpu.VMEM((per_w,), jnp.int32),
            pltpu.VMEM((2, ch, d), jnp.float32),
            pltpu.SemaphoreType.DMA,
            pltpu.SemaphoreType.DMA,
        ],
    )
    def k(table_hbm, idx_hbm, out_hbm, idx_v, buf, gsem, ssem):
        wid = lax.axis_index("s") * nc + lax.axis_index("c")
        base = wid * per_w
        pltpu.sync_copy(idx_hbm.at[pl.ds(base, per_w)], idx_v)

        def start_gather(c, slot):
            pltpu.async_copy(
                table_hbm.at[idx_v.at[pl.ds(c * ch, ch)]], buf.at[slot], gsem)

        def wait_gather(c, slot):
            pltpu.make_async_copy(
                table_hbm.at[idx_v.at[pl.ds(c * ch, ch)]], buf.at[slot],
                gsem).wait()

        def start_scatter(c, slot):
            pltpu.async_copy(
                buf.at[slot], out_hbm.at[pl.ds(base + c * ch, ch)], ssem)

        def wait_scatter():
            pltpu.make_async_copy(
                buf.at[0], out_hbm.at[pl.ds(base, ch)], ssem).wait()

        def start_gather_tail(slot):
            pltpu.async_copy(
                table_hbm.at[idx_v.at[pl.ds(n_ch * ch, tail)]],
                buf.at[slot].at[pl.ds(0, tail)], gsem)

        def wait_gather_tail(slot):
            pltpu.make_async_copy(
                table_hbm.at[idx_v.at[pl.ds(n_ch * ch, tail)]],
                buf.at[slot].at[pl.ds(0, tail)], gsem).wait()

        def start_scatter_tail(slot):
            pltpu.async_copy(
                buf.at[slot].at[pl.ds(0, tail)],
                out_hbm.at[pl.ds(base + n_ch * ch, tail)], ssem)

        def wait_scatter_tail(slot):
            pltpu.make_async_copy(
                buf.at[slot].at[pl.ds(0, tail)],
                out_hbm.at[pl.ds(base + n_ch * ch, tail)], ssem).wait()

        start_gather(0, 0)

        def step(c):
            slot = lax.rem(c, 2)
            wait_gather(c, slot)
            @pl.when(c >= 1)
            def _():
                wait_scatter()

            @pl.when(c + 1 < n_ch)
            def _():
                start_gather(c + 1, 1 - slot)

            if tail:
                @pl.when(c + 1 == n_ch)
                def _():
                    start_gather_tail(1 - slot)

            start_scatter(c, slot)

        pl.loop(0, n_ch)(step)
        if tail:
            slot_t = n_ch % 2
            wait_gather_tail(slot_t)
            wait_scatter()
            start_scatter_tail(slot_t)
            wait_scatter_tail(slot_t)
        else:
            wait_scatter()

    return k(table, idx_flat)


def _tc_ln_chunk(out_prev, tok_t, pos_chunk, seg_table, gamma, beta,
                 off, max_sent, s_total, sb):
    s_c, b, d = tok_t.shape
    assert s_c % sb == 0 and off % sb == 0

    def body(*refs):
        if out_prev is None:
            tok_ref, pos_ref, seg_ref, g_ref, b_ref, o_ref = refs
        else:
            _, tok_ref, pos_ref, seg_ref, g_ref, b_ref, o_ref = refs
        i = pl.program_id(0)
        h = tok_ref[...] + pos_ref[...]
        srow = lax.broadcasted_iota(jnp.int32, (sb, 1, 1), 0) + i * sb + off
        segv = jnp.where(srow < max_sent + 1, seg_ref[0][None, None, :],
                         seg_ref[1][None, None, :])
        h = h + segv
        mean = jnp.mean(h, axis=-1, keepdims=True)
        c = h - mean
        var = jnp.mean(c * c, axis=-1, keepdims=True)
        o_ref[...] = (c * lax.rsqrt(var + 1e-5)) * g_ref[...] + b_ref[...]

    in_specs = [
        pl.BlockSpec((sb, b, d), lambda i: (i, 0, 0)),
        pl.BlockSpec((sb, 1, d), lambda i: (i, 0, 0)),
        pl.BlockSpec((2, d), lambda i: (0, 0)),
        pl.BlockSpec((d,), lambda i: (0,)),
        pl.BlockSpec((d,), lambda i: (0,)),
    ]
    args = [tok_t, pos_chunk.reshape(s_c, 1, d), seg_table, gamma, beta]
    aliases = {}
    if out_prev is not None:
        in_specs = [pl.BlockSpec(memory_space=pl.ANY)] + in_specs
        args = [out_prev] + args
        aliases = {0: 0}

    return pl.pallas_call(
        body,
        grid=(s_c // sb,),
        in_specs=in_specs,
        out_specs=pl.BlockSpec(
            (sb, b, d), lambda i, _o=off // sb: (i + _o, 0, 0)),
        out_shape=jax.ShapeDtypeStruct((s_total, b, d), jnp.float32),
        input_output_aliases=aliases,
    )(*args)


def kernel(x, token_table, pos_table, seg_table, gamma, beta):
    b, s = x.shape
    d = token_table.shape[1]
    max_sent = (s - 2) // 2
    idx_t = x.T.astype(jnp.int32).reshape(-1)
    assert sum(_CHUNKS) == s

    ch = 384
    sb = 6

    out = None
    off = 0
    for s_c in _CHUNKS:
        tok = _sc_gather(token_table, idx_t[off * b:(off + s_c) * b], ch)
        out = _tc_ln_chunk(out, tok.reshape(s_c, b, d),
                           pos_table[off:off + s_c], seg_table,
                           gamma, beta, off, max_sent, s, sb)
        off += s_c
    return jnp.transpose(out, (1, 0, 2))


# speedup vs baseline: 1.0255x; 1.0071x over previous
import functools

import jax
import jax.numpy as jnp
from jax import lax
from jax.experimental import pallas as pl
from jax.experimental.pallas import tpu as pltpu
from jax.experimental.pallas import tpu_sc as plsc

_CHUNKS = (36, 120, 120, 126)


def _sc_gather(table, idx_flat, ch):
    n = idx_flat.shape[0]
    d = table.shape[1]
    info = plsc.get_sparse_core_info()
    nc = info.num_cores
    nw = nc * info.num_subcores
    per_w = n // nw
    n_ch = per_w // ch
    tail = per_w - n_ch * ch
    assert per_w * nw == n and n_ch >= 1 and ch % 8 == 0 and tail % 8 == 0

    mesh = plsc.VectorSubcoreMesh(core_axis_name="c", subcore_axis_name="s")

    @functools.partial(
        pl.kernel,
        mesh=mesh,
        out_type=jax.ShapeDtypeStruct((n, d), jnp.float32),
        scratch_types=[
            pltpu.VMEM((per_w,), jnp.int32),
            pltpu.VMEM((2, ch, d), jnp.float32),
            pltpu.SemaphoreType.DMA,
            pltpu.SemaphoreType.DMA,
        ],
    )
    def k(table_hbm, idx_hbm, out_hbm, idx_v, buf, gsem, ssem):
        wid = lax.axis_index("s") * nc + lax.axis_index("c")
        base = wid * per_w
        pltpu.sync_copy(idx_hbm.at[pl.ds(base, per_w)], idx_v)

        def start_gather(c, slot):
            pltpu.async_copy(
                table_hbm.at[idx_v.at[pl.ds(c * ch, ch)]], buf.at[slot], gsem)

        def wait_gather(c, slot):
            pltpu.make_async_copy(
                table_hbm.at[idx_v.at[pl.ds(c * ch, ch)]], buf.at[slot],
                gsem).wait()

        def start_scatter(c, slot):
            pltpu.async_copy(
                buf.at[slot], out_hbm.at[pl.ds(base + c * ch, ch)], ssem)

        def wait_scatter():
            pltpu.make_async_copy(
                buf.at[0], out_hbm.at[pl.ds(base, ch)], ssem).wait()

        def start_gather_tail(slot):
            pltpu.async_copy(
                table_hbm.at[idx_v.at[pl.ds(n_ch * ch, tail)]],
                buf.at[slot].at[pl.ds(0, tail)], gsem)

        def wait_gather_tail(slot):
            pltpu.make_async_copy(
                table_hbm.at[idx_v.at[pl.ds(n_ch * ch, tail)]],
                buf.at[slot].at[pl.ds(0, tail)], gsem).wait()

        def start_scatter_tail(slot):
            pltpu.async_copy(
                buf.at[slot].at[pl.ds(0, tail)],
                out_hbm.at[pl.ds(base + n_ch * ch, tail)], ssem)

        def wait_scatter_tail(slot):
            pltpu.make_async_copy(
                buf.at[slot].at[pl.ds(0, tail)],
                out_hbm.at[pl.ds(base + n_ch * ch, tail)], ssem).wait()

        start_gather(0, 0)

        def step(c):
            slot = lax.rem(c, 2)
            wait_gather(c, slot)
            @pl.when(c >= 1)
            def _():
                wait_scatter()

            @pl.when(c + 1 < n_ch)
            def _():
                start_gather(c + 1, 1 - slot)

            if tail:
                @pl.when(c + 1 == n_ch)
                def _():
                    start_gather_tail(1 - slot)

            start_scatter(c, slot)

        pl.loop(0, n_ch)(step)
        if tail:
            slot_t = n_ch % 2
            wait_gather_tail(slot_t)
            wait_scatter()
            start_scatter_tail(slot_t)
            wait_scatter_tail(slot_t)
        else:
            wait_scatter()

    return k(table, idx_flat)


def _tc_ln_chunk(out_prev, tok_t, pos_chunk, seg_table, gamma, beta,
                 off, max_sent, s_total, sb):
    s_c, b, d = tok_t.shape
    assert s_c % sb == 0 and off % sb == 0

    def body(*refs):
        if out_prev is None:
            tok_ref, pos_ref, seg_ref, g_ref, b_ref, o_ref = refs
        else:
            _, tok_ref, pos_ref, seg_ref, g_ref, b_ref, o_ref = refs
        i = pl.program_id(0)
        h = tok_ref[...] + pos_ref[...]
        srow = lax.broadcasted_iota(jnp.int32, (sb, 1, 1), 0) + i * sb + off
        segv = jnp.where(srow < max_sent + 1, seg_ref[0][None, None, :],
                         seg_ref[1][None, None, :])
        h = h + segv
        mean = jnp.mean(h, axis=-1, keepdims=True)
        c = h - mean
        var = jnp.mean(c * c, axis=-1, keepdims=True)
        o_ref[...] = (c * lax.rsqrt(var + 1e-5)) * g_ref[...] + b_ref[...]

    in_specs = [
        pl.BlockSpec((sb, b, d), lambda i: (i, 0, 0)),
        pl.BlockSpec((sb, 1, d), lambda i: (i, 0, 0)),
        pl.BlockSpec((2, d), lambda i: (0, 0)),
        pl.BlockSpec((d,), lambda i: (0,)),
        pl.BlockSpec((d,), lambda i: (0,)),
    ]
    args = [tok_t, pos_chunk.reshape(s_c, 1, d), seg_table, gamma, beta]
    aliases = {}
    if out_prev is not None:
        in_specs = [pl.BlockSpec(memory_space=pl.ANY)] + in_specs
        args = [out_prev] + args
        aliases = {0: 0}

    return pl.pallas_call(
        body,
        grid=(s_c // sb,),
        in_specs=in_specs,
        out_specs=pl.BlockSpec(
            (sb, b, d), lambda i, _o=off // sb: (i + _o, 0, 0)),
        out_shape=jax.ShapeDtypeStruct((s_total, b, d), jnp.float32),
        input_output_aliases=aliases,
    )(*args)


def kernel(x, token_table, pos_table, seg_table, gamma, beta):
    b, s = x.shape
    d = token_table.shape[1]
    max_sent = (s - 2) // 2
    idx_t = x.T.astype(jnp.int32).reshape(-1)
    assert sum(_CHUNKS) == s

    ch = 384

    out = None
    off = 0
    for s_c in _CHUNKS:
        sb = max(v for v in (12, 6) if s_c % v == 0 and off % v == 0)
        tok = _sc_gather(token_table, idx_t[off * b:(off + s_c) * b], ch)
        out = _tc_ln_chunk(out, tok.reshape(s_c, b, d),
                           pos_table[off:off + s_c], seg_table,
                           gamma, beta, off, max_sent, s, sb)
        off += s_c
    return jnp.transpose(out, (1, 0, 2))
